# baseline (speedup 1.0000x reference)
import functools

import jax
import jax.numpy as jnp
from jax import lax
from jax.experimental import pallas as pl
from jax.experimental.pallas import tpu as pltpu
from jax.experimental.pallas import tpu_sc as plsc

NUM_PIDS = 100000
NUM_CQ = 5000
NUM_FEAT = 128
BATCH = 1024
OIM_SCALAR = 30.0
IGNORE_INDEX = 5554

BLOCK_N = 4000
NUM_BLOCKS = NUM_PIDS // BLOCK_N

_NEG = -1e30
_LOG2E = 1.4426950408889634
_LN2 = 0.6931471805599453


def _lse_kernel(x_ref, lut_ref, cq_ref, ones_ref, ls_ref, m_ref, acc_ref):
    j = pl.program_id(0)

    @pl.when(j == 0)
    def _init():
        m_ref[...] = jnp.full((BATCH, 1), _NEG, jnp.float32)
        acc_ref[...] = jnp.zeros((BATCH, NUM_FEAT), jnp.float32)

    x = x_ref[...] * (OIM_SCALAR * _LOG2E)
    w = lut_ref[...]
    logits = jax.lax.dot_general(
        x, w, (((1,), (1,)), ((), ())),
        preferred_element_type=jnp.float32).astype(jnp.bfloat16)

    m_old = m_ref[...]
    bm = jnp.max(logits, axis=1, keepdims=True).astype(jnp.float32)
    m_new = jnp.maximum(m_old, bm)
    e = jnp.exp2(logits - m_new.astype(jnp.bfloat16))
    part = jax.lax.dot_general(
        e, ones_ref[:BLOCK_N, :], (((1,), (0,)), ((), ())),
        preferred_element_type=jnp.float32)
    acc_ref[...] = acc_ref[...] * jnp.exp2(m_old - m_new) + part
    m_ref[...] = m_new

    @pl.when(j == NUM_BLOCKS - 1)
    def _tail():
        cq = cq_ref[...]
        logits2 = jax.lax.dot_general(
            x, cq, (((1,), (1,)), ((), ())),
            preferred_element_type=jnp.float32).astype(jnp.bfloat16)
        m_old2 = m_ref[...]
        bm2 = jnp.max(logits2, axis=1, keepdims=True).astype(jnp.float32)
        m2 = jnp.maximum(m_old2, bm2)
        e2 = jnp.exp2(logits2 - m2.astype(jnp.bfloat16))
        part2 = jax.lax.dot_general(
            e2, ones_ref[...], (((1,), (0,)), ((), ())),
            preferred_element_type=jnp.float32)
        acc = acc_ref[...] * jnp.exp2(m_old2 - m2) + part2
        s = acc[:, 0:1]
        ls_ref[...] = (m2 + jnp.log2(s)) * _LN2


@functools.cache
def _sc_gather_fn():
    info = plsc.get_sparse_core_info()
    nc, ns = info.num_cores, info.num_subcores
    bpw = BATCH // (nc * ns)
    mesh = plsc.VectorSubcoreMesh(core_axis_name="c", subcore_axis_name="s")

    @functools.partial(
        pl.kernel,
        mesh=mesh,
        out_type=jax.ShapeDtypeStruct((BATCH, NUM_FEAT), jnp.float32),
        scratch_types=[
            pltpu.VMEM((bpw,), jnp.int32),
            pltpu.VMEM((bpw, NUM_FEAT), jnp.float32),
            pltpu.SemaphoreType.DMA,
        ],
    )
    def _sc_gather(lab_hbm, lut_hbm, out_hbm, idx_v, rows_v, sem):
        wid = lax.axis_index("s") * nc + lax.axis_index("c")
        base = wid * bpw
        pltpu.sync_copy(lab_hbm.at[pl.ds(base, bpw)], idx_v)
        for k in range(bpw // 16):
            v = idx_v[pl.ds(k * 16, 16)]
            idx_v[pl.ds(k * 16, 16)] = jnp.maximum(v - 1, 0)
        pltpu.async_copy(lut_hbm.at[idx_v], rows_v, sem).wait()
        pltpu.sync_copy(rows_v, out_hbm.at[pl.ds(base, bpw)])

    return _sc_gather


def _loss_kernel(ls_ref, g_ref, x_ref, lab_ref, out_ref):
    xs = x_ref[...] * OIM_SCALAR
    picked = jnp.sum(xs * g_ref[...], axis=1, keepdims=True)
    lab = lab_ref[...]
    label_all = lab - 1
    valid = jnp.logical_and(label_all >= 0, label_all != IGNORE_INDEX)
    vf = valid.astype(jnp.float32)
    denom = jnp.maximum(jnp.sum(vf), 1.0)
    ce = ls_ref[...] - picked
    out_ref[...] = jnp.sum(ce * vf, keepdims=True).reshape(1, 1) / denom


@jax.jit
def _oim_loss(inputs, roi_label, lut, cq):
    x = inputs.astype(jnp.float32)
    lut = lut.astype(jnp.float32)
    cq = cq.astype(jnp.float32)
    lab2d = roi_label.reshape(BATCH, 1)

    ones = jnp.ones((NUM_CQ, NUM_FEAT), jnp.bfloat16)
    ls = pl.pallas_call(
        _lse_kernel,
        grid=(NUM_BLOCKS,),
        in_specs=[
            pl.BlockSpec((BATCH, NUM_FEAT), lambda j: (0, 0)),
            pl.BlockSpec((BLOCK_N, NUM_FEAT), lambda j: (j, 0)),
            pl.BlockSpec((NUM_CQ, NUM_FEAT), lambda j: (0, 0)),
            pl.BlockSpec((NUM_CQ, NUM_FEAT), lambda j: (0, 0)),
        ],
        out_specs=pl.BlockSpec((BATCH, 1), lambda j: (0, 0)),
        out_shape=jax.ShapeDtypeStruct((BATCH, 1), jnp.float32),
        scratch_shapes=[
            pltpu.VMEM((BATCH, 1), jnp.float32),
            pltpu.VMEM((BATCH, NUM_FEAT), jnp.float32),
        ],
    )(x, lut, cq, ones)

    g = _sc_gather_fn()(roi_label, lut)

    out = pl.pallas_call(
        _loss_kernel,
        in_specs=[
            pl.BlockSpec((BATCH, 1), lambda: (0, 0)),
            pl.BlockSpec((BATCH, NUM_FEAT), lambda: (0, 0)),
            pl.BlockSpec((BATCH, NUM_FEAT), lambda: (0, 0)),
            pl.BlockSpec((BATCH, 1), lambda: (0, 0)),
        ],
        out_specs=pl.BlockSpec((1, 1), lambda: (0, 0)),
        out_shape=jax.ShapeDtypeStruct((1, 1), jnp.float32),
    )(ls, g, x, lab2d)
    return out.reshape(())


def kernel(inputs, roi_label, lut, cq):
    return _oim_loss(inputs, roi_label, lut, cq)


# speedup vs baseline: 1.0058x; 1.0058x over previous
import functools

import jax
import jax.numpy as jnp
from jax import lax
from jax.experimental import pallas as pl
from jax.experimental.pallas import tpu as pltpu
from jax.experimental.pallas import tpu_sc as plsc

NUM_PIDS = 100000
NUM_CQ = 5000
NUM_FEAT = 128
BATCH = 1024
OIM_SCALAR = 30.0
IGNORE_INDEX = 5554

BLOCK_N = 4000
NUM_BLOCKS = NUM_PIDS // BLOCK_N

_NEG = -1e30
_LOG2E = 1.4426950408889634
_LN2 = 0.6931471805599453


def _lse_kernel(x_ref, lut_ref, cq_ref, ones_ref, ls_ref, m_ref, acc_ref):
    j = pl.program_id(0)

    @pl.when(j == 0)
    def _init():
        m_ref[...] = jnp.full((BATCH, 1), _NEG, jnp.float32)
        acc_ref[...] = jnp.zeros((BATCH, NUM_FEAT), jnp.float32)

    x = (x_ref[...] * (OIM_SCALAR * _LOG2E)).astype(jnp.bfloat16)
    w = lut_ref[...].astype(jnp.bfloat16)
    logits = jax.lax.dot_general(
        x, w, (((1,), (1,)), ((), ())),
        preferred_element_type=jnp.float32).astype(jnp.bfloat16)

    m_old = m_ref[...]
    bm = jnp.max(logits, axis=1, keepdims=True).astype(jnp.float32)
    m_new = jnp.maximum(m_old, bm)
    e = jnp.exp2(logits - m_new.astype(jnp.bfloat16))
    part = jax.lax.dot_general(
        e, ones_ref[:BLOCK_N, :], (((1,), (0,)), ((), ())),
        preferred_element_type=jnp.float32)
    acc_ref[...] = acc_ref[...] * jnp.exp2(m_old - m_new) + part
    m_ref[...] = m_new

    @pl.when(j == NUM_BLOCKS - 1)
    def _tail():
        cq = cq_ref[...].astype(jnp.bfloat16)
        logits2 = jax.lax.dot_general(
            x, cq, (((1,), (1,)), ((), ())),
            preferred_element_type=jnp.float32).astype(jnp.bfloat16)
        m_old2 = m_ref[...]
        bm2 = jnp.max(logits2, axis=1, keepdims=True).astype(jnp.float32)
        m2 = jnp.maximum(m_old2, bm2)
        e2 = jnp.exp2(logits2 - m2.astype(jnp.bfloat16))
        part2 = jax.lax.dot_general(
            e2, ones_ref[...], (((1,), (0,)), ((), ())),
            preferred_element_type=jnp.float32)
        acc = acc_ref[...] * jnp.exp2(m_old2 - m2) + part2
        s = acc[:, 0:1]
        ls_ref[...] = (m2 + jnp.log2(s)) * _LN2


@functools.cache
def _sc_gather_fn():
    info = plsc.get_sparse_core_info()
    nc, ns = info.num_cores, info.num_subcores
    bpw = BATCH // (nc * ns)
    mesh = plsc.VectorSubcoreMesh(core_axis_name="c", subcore_axis_name="s")

    @functools.partial(
        pl.kernel,
        mesh=mesh,
        out_type=jax.ShapeDtypeStruct((BATCH, NUM_FEAT), jnp.float32),
        scratch_types=[
            pltpu.VMEM((bpw,), jnp.int32),
            pltpu.VMEM((bpw, NUM_FEAT), jnp.float32),
            pltpu.SemaphoreType.DMA,
        ],
    )
    def _sc_gather(lab_hbm, lut_hbm, out_hbm, idx_v, rows_v, sem):
        wid = lax.axis_index("s") * nc + lax.axis_index("c")
        base = wid * bpw
        pltpu.sync_copy(lab_hbm.at[pl.ds(base, bpw)], idx_v)
        for k in range(bpw // 16):
            v = idx_v[pl.ds(k * 16, 16)]
            idx_v[pl.ds(k * 16, 16)] = jnp.maximum(v - 1, 0)
        pltpu.async_copy(lut_hbm.at[idx_v], rows_v, sem).wait()
        pltpu.sync_copy(rows_v, out_hbm.at[pl.ds(base, bpw)])

    return _sc_gather


def _loss_kernel(ls_ref, g_ref, x_ref, lab_ref, out_ref):
    xs = x_ref[...] * OIM_SCALAR
    picked = jnp.sum(xs * g_ref[...], axis=1, keepdims=True)
    lab = lab_ref[...]
    label_all = lab - 1
    valid = jnp.logical_and(label_all >= 0, label_all != IGNORE_INDEX)
    vf = valid.astype(jnp.float32)
    denom = jnp.maximum(jnp.sum(vf), 1.0)
    ce = ls_ref[...] - picked
    out_ref[...] = jnp.sum(ce * vf, keepdims=True).reshape(1, 1) / denom


@jax.jit
def _oim_loss(inputs, roi_label, lut, cq):
    x = inputs.astype(jnp.float32)
    lut = lut.astype(jnp.float32)
    cq = cq.astype(jnp.float32)
    lab2d = roi_label.reshape(BATCH, 1)

    ones = jnp.ones((NUM_CQ, NUM_FEAT), jnp.bfloat16)
    ls = pl.pallas_call(
        _lse_kernel,
        grid=(NUM_BLOCKS,),
        in_specs=[
            pl.BlockSpec((BATCH, NUM_FEAT), lambda j: (0, 0)),
            pl.BlockSpec((BLOCK_N, NUM_FEAT), lambda j: (j, 0)),
            pl.BlockSpec((NUM_CQ, NUM_FEAT), lambda j: (0, 0)),
            pl.BlockSpec((NUM_CQ, NUM_FEAT), lambda j: (0, 0)),
        ],
        out_specs=pl.BlockSpec((BATCH, 1), lambda j: (0, 0)),
        out_shape=jax.ShapeDtypeStruct((BATCH, 1), jnp.float32),
        scratch_shapes=[
            pltpu.VMEM((BATCH, 1), jnp.float32),
            pltpu.VMEM((BATCH, NUM_FEAT), jnp.float32),
        ],
    )(x, lut, cq, ones)

    g = _sc_gather_fn()(roi_label, lut)

    out = pl.pallas_call(
        _loss_kernel,
        in_specs=[
            pl.BlockSpec((BATCH, 1), lambda: (0, 0)),
            pl.BlockSpec((BATCH, NUM_FEAT), lambda: (0, 0)),
            pl.BlockSpec((BATCH, NUM_FEAT), lambda: (0, 0)),
            pl.BlockSpec((BATCH, 1), lambda: (0, 0)),
        ],
        out_specs=pl.BlockSpec((1, 1), lambda: (0, 0)),
        out_shape=jax.ShapeDtypeStruct((1, 1), jnp.float32),
    )(ls, g, x, lab2d)
    return out.reshape(())


def kernel(inputs, roi_label, lut, cq):
    return _oim_loss(inputs, roi_label, lut, cq)


# speedup vs baseline: 1.1302x; 1.1237x over previous
import functools

import jax
import jax.numpy as jnp
from jax import lax
from jax.experimental import pallas as pl
from jax.experimental.pallas import tpu as pltpu
from jax.experimental.pallas import tpu_sc as plsc

NUM_PIDS = 100000
NUM_CQ = 5000
NUM_FEAT = 128
BATCH = 1024
OIM_SCALAR = 30.0
IGNORE_INDEX = 5554

BLOCK_N = 4000
NUM_BLOCKS = NUM_PIDS // BLOCK_N

_NEG = -1e30
_LOG2E = 1.4426950408889634
_LN2 = 0.6931471805599453


def _lse_kernel(x_ref, lut_ref, cq_ref, ls_ref, m_ref, s_ref):
    j = pl.program_id(0)

    @pl.when(j == 0)
    def _init():
        m_ref[...] = jnp.full((BATCH, 1), _NEG, jnp.float32)
        s_ref[...] = jnp.zeros((BATCH, 1), jnp.float32)

    x = (x_ref[...] * (OIM_SCALAR * _LOG2E)).astype(jnp.bfloat16)
    w = lut_ref[...].astype(jnp.bfloat16)
    logits = jax.lax.dot_general(
        x, w, (((1,), (1,)), ((), ())),
        preferred_element_type=jnp.float32)

    m_old = m_ref[...]
    bm = jnp.max(logits, axis=1, keepdims=True)
    m_new = jnp.maximum(m_old, bm)
    e = jnp.exp2(logits - m_new)
    part = jnp.sum(e, axis=1, keepdims=True)
    s_ref[...] = s_ref[...] * jnp.exp2(m_old - m_new) + part
    m_ref[...] = m_new

    @pl.when(j == NUM_BLOCKS - 1)
    def _tail():
        cq = cq_ref[...].astype(jnp.bfloat16)
        logits2 = jax.lax.dot_general(
            x, cq, (((1,), (1,)), ((), ())),
            preferred_element_type=jnp.float32)
        m_old2 = m_ref[...]
        bm2 = jnp.max(logits2, axis=1, keepdims=True)
        m2 = jnp.maximum(m_old2, bm2)
        e2 = jnp.exp2(logits2 - m2)
        part2 = jnp.sum(e2, axis=1, keepdims=True)
        s = s_ref[...] * jnp.exp2(m_old2 - m2) + part2
        ls_ref[...] = (m2 + jnp.log2(s)) * _LN2


@functools.cache
def _sc_gather_fn():
    info = plsc.get_sparse_core_info()
    nc, ns = info.num_cores, info.num_subcores
    bpw = BATCH // (nc * ns)
    mesh = plsc.VectorSubcoreMesh(core_axis_name="c", subcore_axis_name="s")

    @functools.partial(
        pl.kernel,
        mesh=mesh,
        out_type=jax.ShapeDtypeStruct((BATCH, NUM_FEAT), jnp.float32),
        scratch_types=[
            pltpu.VMEM((bpw,), jnp.int32),
            pltpu.VMEM((bpw, NUM_FEAT), jnp.float32),
            pltpu.SemaphoreType.DMA,
        ],
    )
    def _sc_gather(lab_hbm, lut_hbm, out_hbm, idx_v, rows_v, sem):
        wid = lax.axis_index("s") * nc + lax.axis_index("c")
        base = wid * bpw
        pltpu.sync_copy(lab_hbm.at[pl.ds(base, bpw)], idx_v)
        for k in range(bpw // 16):
            v = idx_v[pl.ds(k * 16, 16)]
            idx_v[pl.ds(k * 16, 16)] = jnp.maximum(v - 1, 0)
        pltpu.async_copy(lut_hbm.at[idx_v], rows_v, sem).wait()
        pltpu.sync_copy(rows_v, out_hbm.at[pl.ds(base, bpw)])

    return _sc_gather


def _loss_kernel(ls_ref, g_ref, x_ref, lab_ref, out_ref):
    xs = x_ref[...] * OIM_SCALAR
    picked = jnp.sum(xs * g_ref[...], axis=1, keepdims=True)
    lab = lab_ref[...]
    label_all = lab - 1
    valid = jnp.logical_and(label_all >= 0, label_all != IGNORE_INDEX)
    vf = valid.astype(jnp.float32)
    denom = jnp.maximum(jnp.sum(vf), 1.0)
    ce = ls_ref[...] - picked
    out_ref[...] = jnp.sum(ce * vf, keepdims=True).reshape(1, 1) / denom


@jax.jit
def _oim_loss(inputs, roi_label, lut, cq):
    x = inputs.astype(jnp.float32)
    lut = lut.astype(jnp.float32)
    cq = cq.astype(jnp.float32)
    lab2d = roi_label.reshape(BATCH, 1)

    ls = pl.pallas_call(
        _lse_kernel,
        grid=(NUM_BLOCKS,),
        in_specs=[
            pl.BlockSpec((BATCH, NUM_FEAT), lambda j: (0, 0)),
            pl.BlockSpec((BLOCK_N, NUM_FEAT), lambda j: (j, 0)),
            pl.BlockSpec((NUM_CQ, NUM_FEAT), lambda j: (0, 0)),
        ],
        out_specs=pl.BlockSpec((BATCH, 1), lambda j: (0, 0)),
        out_shape=jax.ShapeDtypeStruct((BATCH, 1), jnp.float32),
        scratch_shapes=[
            pltpu.VMEM((BATCH, 1), jnp.float32),
            pltpu.VMEM((BATCH, 1), jnp.float32),
        ],
    )(x, lut, cq)

    g = _sc_gather_fn()(roi_label, lut)

    out = pl.pallas_call(
        _loss_kernel,
        in_specs=[
            pl.BlockSpec((BATCH, 1), lambda: (0, 0)),
            pl.BlockSpec((BATCH, NUM_FEAT), lambda: (0, 0)),
            pl.BlockSpec((BATCH, NUM_FEAT), lambda: (0, 0)),
            pl.BlockSpec((BATCH, 1), lambda: (0, 0)),
        ],
        out_specs=pl.BlockSpec((1, 1), lambda: (0, 0)),
        out_shape=jax.ShapeDtypeStruct((1, 1), jnp.float32),
    )(ls, g, x, lab2d)
    return out.reshape(())


def kernel(inputs, roi_label, lut, cq):
    return _oim_loss(inputs, roi_label, lut, cq)


# speedup vs baseline: 1.6463x; 1.4566x over previous
import functools

import jax
import jax.numpy as jnp
from jax import lax
from jax.experimental import pallas as pl
from jax.experimental.pallas import tpu as pltpu
from jax.experimental.pallas import tpu_sc as plsc

NUM_PIDS = 100000
NUM_CQ = 5000
NUM_FEAT = 128
BATCH = 1024
OIM_SCALAR = 30.0
IGNORE_INDEX = 5554

BLOCK_N = 4000
NUM_BLOCKS = NUM_PIDS // BLOCK_N

_NEG = -1e30
_LOG2E = 1.4426950408889634
_LN2 = 0.6931471805599453


def _lse_kernel(x_ref, lut_ref, cq_ref, ls_ref, m_ref, s_ref):
    j = pl.program_id(0)

    @pl.when(j == 0)
    def _init():
        m_ref[...] = jnp.full((BATCH, 1), _NEG, jnp.float32)
        s_ref[...] = jnp.zeros((BATCH, 1), jnp.float32)

    x = (x_ref[...] * (OIM_SCALAR * _LOG2E)).astype(jnp.bfloat16)
    w = lut_ref[...].astype(jnp.bfloat16)
    logits = jax.lax.dot_general(
        x, w, (((1,), (1,)), ((), ())),
        preferred_element_type=jnp.float32)

    m_old = m_ref[...]
    part = jnp.sum(jnp.exp2(logits - m_old), axis=1, keepdims=True)
    s_try = s_ref[...] + part
    ok = jnp.all(jnp.isfinite(s_try))

    @pl.when(ok)
    def _commit():
        s_ref[...] = s_try

    @pl.when(jnp.logical_not(ok))
    def _redo():
        bm = jnp.max(logits, axis=1, keepdims=True)
        m_new = jnp.maximum(m_old, bm)
        e = jnp.exp2(logits - m_new)
        s_ref[...] = (s_ref[...] * jnp.exp2(m_old - m_new)
                      + jnp.sum(e, axis=1, keepdims=True))
        m_ref[...] = m_new

    @pl.when(j == NUM_BLOCKS - 1)
    def _tail():
        cq = cq_ref[...].astype(jnp.bfloat16)
        logits2 = jax.lax.dot_general(
            x, cq, (((1,), (1,)), ((), ())),
            preferred_element_type=jnp.float32)
        m_old2 = m_ref[...]
        bm2 = jnp.max(logits2, axis=1, keepdims=True)
        m2 = jnp.maximum(m_old2, bm2)
        e2 = jnp.exp2(logits2 - m2)
        part2 = jnp.sum(e2, axis=1, keepdims=True)
        s = s_ref[...] * jnp.exp2(m_old2 - m2) + part2
        ls_ref[...] = (m2 + jnp.log2(s)) * _LN2


@functools.cache
def _sc_gather_fn():
    info = plsc.get_sparse_core_info()
    nc, ns = info.num_cores, info.num_subcores
    bpw = BATCH // (nc * ns)
    mesh = plsc.VectorSubcoreMesh(core_axis_name="c", subcore_axis_name="s")

    @functools.partial(
        pl.kernel,
        mesh=mesh,
        out_type=jax.ShapeDtypeStruct((BATCH, NUM_FEAT), jnp.float32),
        scratch_types=[
            pltpu.VMEM((bpw,), jnp.int32),
            pltpu.VMEM((bpw, NUM_FEAT), jnp.float32),
            pltpu.SemaphoreType.DMA,
        ],
    )
    def _sc_gather(lab_hbm, lut_hbm, out_hbm, idx_v, rows_v, sem):
        wid = lax.axis_index("s") * nc + lax.axis_index("c")
        base = wid * bpw
        pltpu.sync_copy(lab_hbm.at[pl.ds(base, bpw)], idx_v)
        for k in range(bpw // 16):
            v = idx_v[pl.ds(k * 16, 16)]
            idx_v[pl.ds(k * 16, 16)] = jnp.maximum(v - 1, 0)
        pltpu.async_copy(lut_hbm.at[idx_v], rows_v, sem).wait()
        pltpu.sync_copy(rows_v, out_hbm.at[pl.ds(base, bpw)])

    return _sc_gather


def _loss_kernel(ls_ref, g_ref, x_ref, lab_ref, out_ref):
    xs = x_ref[...] * OIM_SCALAR
    picked = jnp.sum(xs * g_ref[...], axis=1, keepdims=True)
    lab = lab_ref[...]
    label_all = lab - 1
    valid = jnp.logical_and(label_all >= 0, label_all != IGNORE_INDEX)
    vf = valid.astype(jnp.float32)
    denom = jnp.maximum(jnp.sum(vf), 1.0)
    ce = ls_ref[...] - picked
    out_ref[...] = jnp.sum(ce * vf, keepdims=True).reshape(1, 1) / denom


@jax.jit
def _oim_loss(inputs, roi_label, lut, cq):
    x = inputs.astype(jnp.float32)
    lut = lut.astype(jnp.float32)
    cq = cq.astype(jnp.float32)
    lab2d = roi_label.reshape(BATCH, 1)

    ls = pl.pallas_call(
        _lse_kernel,
        grid=(NUM_BLOCKS,),
        in_specs=[
            pl.BlockSpec((BATCH, NUM_FEAT), lambda j: (0, 0)),
            pl.BlockSpec((BLOCK_N, NUM_FEAT), lambda j: (j, 0)),
            pl.BlockSpec((NUM_CQ, NUM_FEAT), lambda j: (0, 0)),
        ],
        out_specs=pl.BlockSpec((BATCH, 1), lambda j: (0, 0)),
        out_shape=jax.ShapeDtypeStruct((BATCH, 1), jnp.float32),
        scratch_shapes=[
            pltpu.VMEM((BATCH, 1), jnp.float32),
            pltpu.VMEM((BATCH, 1), jnp.float32),
        ],
    )(x, lut, cq)

    g = _sc_gather_fn()(roi_label, lut)

    out = pl.pallas_call(
        _loss_kernel,
        in_specs=[
            pl.BlockSpec((BATCH, 1), lambda: (0, 0)),
            pl.BlockSpec((BATCH, NUM_FEAT), lambda: (0, 0)),
            pl.BlockSpec((BATCH, NUM_FEAT), lambda: (0, 0)),
            pl.BlockSpec((BATCH, 1), lambda: (0, 0)),
        ],
        out_specs=pl.BlockSpec((1, 1), lambda: (0, 0)),
        out_shape=jax.ShapeDtypeStruct((1, 1), jnp.float32),
    )(ls, g, x, lab2d)
    return out.reshape(())


def kernel(inputs, roi_label, lut, cq):
    return _oim_loss(inputs, roi_label, lut, cq)


# speedup vs baseline: 1.6831x; 1.0224x over previous
import functools

import jax
import jax.numpy as jnp
from jax import lax
from jax.experimental import pallas as pl
from jax.experimental.pallas import tpu as pltpu
from jax.experimental.pallas import tpu_sc as plsc

NUM_PIDS = 100000
NUM_CQ = 5000
NUM_FEAT = 128
BATCH = 1024
OIM_SCALAR = 30.0
IGNORE_INDEX = 5554

BLOCK_N = 5000
NUM_BLOCKS = NUM_PIDS // BLOCK_N

_NEG = -1e30
_LOG2E = 1.4426950408889634
_LN2 = 0.6931471805599453


def _lse_kernel(x_ref, lut_ref, cq_ref, ls_ref, m_ref, s_ref):
    j = pl.program_id(0)

    @pl.when(j == 0)
    def _init():
        m_ref[...] = jnp.full((BATCH, 1), _NEG, jnp.float32)
        s_ref[...] = jnp.zeros((BATCH, 1), jnp.float32)

    x = (x_ref[...] * (OIM_SCALAR * _LOG2E)).astype(jnp.bfloat16)
    w = lut_ref[...].astype(jnp.bfloat16)
    logits = jax.lax.dot_general(
        x, w, (((1,), (1,)), ((), ())),
        preferred_element_type=jnp.float32)

    m_old = m_ref[...]
    part = jnp.sum(jnp.exp2(logits - m_old), axis=1, keepdims=True)
    s_try = s_ref[...] + part
    ok = jnp.all(jnp.isfinite(s_try))

    @pl.when(ok)
    def _commit():
        s_ref[...] = s_try

    @pl.when(jnp.logical_not(ok))
    def _redo():
        bm = jnp.max(logits, axis=1, keepdims=True)
        m_new = jnp.maximum(m_old, bm)
        e = jnp.exp2(logits - m_new)
        s_ref[...] = (s_ref[...] * jnp.exp2(m_old - m_new)
                      + jnp.sum(e, axis=1, keepdims=True))
        m_ref[...] = m_new

    @pl.when(j == NUM_BLOCKS - 1)
    def _tail():
        cq = cq_ref[...].astype(jnp.bfloat16)
        logits2 = jax.lax.dot_general(
            x, cq, (((1,), (1,)), ((), ())),
            preferred_element_type=jnp.float32)
        m_old2 = m_ref[...]
        bm2 = jnp.max(logits2, axis=1, keepdims=True)
        m2 = jnp.maximum(m_old2, bm2)
        e2 = jnp.exp2(logits2 - m2)
        part2 = jnp.sum(e2, axis=1, keepdims=True)
        s = s_ref[...] * jnp.exp2(m_old2 - m2) + part2
        ls_ref[...] = (m2 + jnp.log2(s)) * _LN2


@functools.cache
def _sc_gather_fn():
    info = plsc.get_sparse_core_info()
    nc, ns = info.num_cores, info.num_subcores
    bpw = BATCH // (nc * ns)
    mesh = plsc.VectorSubcoreMesh(core_axis_name="c", subcore_axis_name="s")

    @functools.partial(
        pl.kernel,
        mesh=mesh,
        out_type=jax.ShapeDtypeStruct((BATCH, NUM_FEAT), jnp.float32),
        scratch_types=[
            pltpu.VMEM((bpw,), jnp.int32),
            pltpu.VMEM((bpw, NUM_FEAT), jnp.float32),
            pltpu.SemaphoreType.DMA,
        ],
    )
    def _sc_gather(lab_hbm, lut_hbm, out_hbm, idx_v, rows_v, sem):
        wid = lax.axis_index("s") * nc + lax.axis_index("c")
        base = wid * bpw
        pltpu.sync_copy(lab_hbm.at[pl.ds(base, bpw)], idx_v)
        for k in range(bpw // 16):
            v = idx_v[pl.ds(k * 16, 16)]
            idx_v[pl.ds(k * 16, 16)] = jnp.maximum(v - 1, 0)
        pltpu.async_copy(lut_hbm.at[idx_v], rows_v, sem).wait()
        pltpu.sync_copy(rows_v, out_hbm.at[pl.ds(base, bpw)])

    return _sc_gather


def _loss_kernel(ls_ref, g_ref, x_ref, lab_ref, out_ref):
    xs = x_ref[...] * OIM_SCALAR
    picked = jnp.sum(xs * g_ref[...], axis=1, keepdims=True)
    lab = lab_ref[...]
    label_all = lab - 1
    valid = jnp.logical_and(label_all >= 0, label_all != IGNORE_INDEX)
    vf = valid.astype(jnp.float32)
    denom = jnp.maximum(jnp.sum(vf), 1.0)
    ce = ls_ref[...] - picked
    out_ref[...] = jnp.sum(ce * vf, keepdims=True).reshape(1, 1) / denom


@jax.jit
def _oim_loss(inputs, roi_label, lut, cq):
    x = inputs.astype(jnp.float32)
    lut = lut.astype(jnp.float32)
    cq = cq.astype(jnp.float32)
    lab2d = roi_label.reshape(BATCH, 1)

    ls = pl.pallas_call(
        _lse_kernel,
        grid=(NUM_BLOCKS,),
        in_specs=[
            pl.BlockSpec((BATCH, NUM_FEAT), lambda j: (0, 0)),
            pl.BlockSpec((BLOCK_N, NUM_FEAT), lambda j: (j, 0)),
            pl.BlockSpec((NUM_CQ, NUM_FEAT), lambda j: (0, 0)),
        ],
        out_specs=pl.BlockSpec((BATCH, 1), lambda j: (0, 0)),
        out_shape=jax.ShapeDtypeStruct((BATCH, 1), jnp.float32),
        scratch_shapes=[
            pltpu.VMEM((BATCH, 1), jnp.float32),
            pltpu.VMEM((BATCH, 1), jnp.float32),
        ],
    )(x, lut, cq)

    g = _sc_gather_fn()(roi_label, lut)

    out = pl.pallas_call(
        _loss_kernel,
        in_specs=[
            pl.BlockSpec((BATCH, 1), lambda: (0, 0)),
            pl.BlockSpec((BATCH, NUM_FEAT), lambda: (0, 0)),
            pl.BlockSpec((BATCH, NUM_FEAT), lambda: (0, 0)),
            pl.BlockSpec((BATCH, 1), lambda: (0, 0)),
        ],
        out_specs=pl.BlockSpec((1, 1), lambda: (0, 0)),
        out_shape=jax.ShapeDtypeStruct((1, 1), jnp.float32),
    )(ls, g, x, lab2d)
    return out.reshape(())


def kernel(inputs, roi_label, lut, cq):
    return _oim_loss(inputs, roi_label, lut, cq)
